# speedup vs baseline: 1.7111x; 1.7111x over previous
import jax
import jax.numpy as jnp
from jax import lax
from jax.experimental import pallas as pl
from jax.experimental.pallas import tpu as pltpu
from jax.experimental.pallas import tpu_sc as plsc

_NC = 2
_NS = 16
_NW = _NC * _NS

_V = 32768
_D = 1024
_B = 4 * 8192
_BPW = _B // _NW
_C = 32
_NCHUNK = _BPW // _C


def _gather_body(table_hbm, idx_hbm, out_hbm, idx_v, buf, gsem, psem):
    wid = lax.axis_index("s") * _NC + lax.axis_index("c")
    base = wid * _BPW
    pltpu.sync_copy(idx_hbm.at[wid], idx_v)

    def gather(j):
        return pltpu.make_async_copy(
            table_hbm.at[idx_v.at[j]], buf.at[j % 2], gsem.at[j % 2])

    def put(j):
        return pltpu.make_async_copy(
            buf.at[j % 2], out_hbm.at[pl.ds(base + j * _C, _C)],
            psem.at[j % 2])

    gather(0).start()
    for j in range(_NCHUNK):
        if j + 1 < _NCHUNK:
            if j >= 1:
                put(j - 1).wait()
            gather(j + 1).start()
        gather(j).wait()
        put(j).start()
    put(_NCHUNK - 2).wait()
    put(_NCHUNK - 1).wait()


def _run(encodes, xw):
    mesh = plsc.VectorSubcoreMesh(core_axis_name="c", subcore_axis_name="s")
    fn = pl.kernel(
        _gather_body,
        out_type=jax.ShapeDtypeStruct((_B, _D), jnp.float32),
        mesh=mesh,
        scratch_types=[
            pltpu.VMEM((_NCHUNK, _C), jnp.int32),
            pltpu.VMEM((2, _C, _D), jnp.float32),
            pltpu.SemaphoreType.DMA((2,)),
            pltpu.SemaphoreType.DMA((2,)),
        ],
    )
    return fn(encodes, xw)


def kernel(encodes, X):
    xw = X.reshape(_NW, _NCHUNK, _C).astype(jnp.int32)
    out = _run(encodes, xw)
    return out.reshape(X.shape + (_D,))


# speedup vs baseline: 1.7144x; 1.0020x over previous
import jax
import jax.numpy as jnp
from jax import lax
from jax.experimental import pallas as pl
from jax.experimental.pallas import tpu as pltpu
from jax.experimental.pallas import tpu_sc as plsc

_NC = 2
_NS = 16
_NW = _NC * _NS

_V = 32768
_D = 1024
_B = 4 * 8192
_BPW = _B // _NW
_C = 32
_NCHUNK = _BPW // _C
_NBUF = 3


def _gather_body(table_hbm, idx_hbm, out_hbm, idx_v, buf, gsem, psem):
    wid = lax.axis_index("s") * _NC + lax.axis_index("c")
    base = wid * _BPW
    pltpu.sync_copy(idx_hbm.at[wid], idx_v)

    def gather(j):
        return pltpu.make_async_copy(
            table_hbm.at[idx_v.at[j]], buf.at[j % _NBUF], gsem.at[j % _NBUF])

    def put(j):
        return pltpu.make_async_copy(
            buf.at[j % _NBUF], out_hbm.at[pl.ds(base + j * _C, _C)],
            psem.at[j % _NBUF])

    for j in range(_NBUF - 1):
        gather(j).start()
    for j in range(_NCHUNK):
        gather(j).wait()
        put(j).start()
        nxt = j + _NBUF - 1
        if nxt < _NCHUNK:
            if j >= 1:
                put(j - 1).wait()
            gather(nxt).start()
    for j in range(_NCHUNK - _NBUF, _NCHUNK):
        put(j).wait()


def _run(encodes, xw):
    mesh = plsc.VectorSubcoreMesh(core_axis_name="c", subcore_axis_name="s")
    fn = pl.kernel(
        _gather_body,
        out_type=jax.ShapeDtypeStruct((_B, _D), jnp.float32),
        mesh=mesh,
        scratch_types=[
            pltpu.VMEM((_NCHUNK, _C), jnp.int32),
            pltpu.VMEM((_NBUF, _C, _D), jnp.float32),
            pltpu.SemaphoreType.DMA((_NBUF,)),
            pltpu.SemaphoreType.DMA((_NBUF,)),
        ],
    )
    return fn(encodes, xw)


def kernel(encodes, X):
    xw = X.reshape(_NW, _NCHUNK, _C).astype(jnp.int32)
    out = _run(encodes, xw)
    return out.reshape(X.shape + (_D,))
